# baseline (speedup 1.0000x reference)
import functools

import jax
import jax.numpy as jnp
from jax import lax
from jax.experimental import pallas as pl
from jax.experimental.pallas import tpu as pltpu
from jax.experimental.pallas import tpu_sc as plsc

_NC = 2
_NS = 16
_NW = _NC * _NS

_CHUNK = 56
_NBUF = 2


@functools.partial(jax.jit, static_argnums=(2, 3))
def _sc_gather(embedding, idx_flat, N, D):
    b_per_w = N // _NW
    nchunks = -(-b_per_w // _CHUNK)
    mesh = plsc.VectorSubcoreMesh(core_axis_name="c", subcore_axis_name="s")

    @functools.partial(
        pl.kernel,
        out_type=jax.ShapeDtypeStruct((N, D), jnp.float32),
        mesh=mesh,
        scratch_types=(
            [pltpu.VMEM((b_per_w,), jnp.int32)]
            + [pltpu.VMEM((_CHUNK, D), jnp.float32) for _ in range(_NBUF)]
            + [pltpu.SemaphoreType.DMA for _ in range(2 * _NBUF)]
        ),
    )
    def gather_kernel(table_hbm, idx_hbm, out_hbm, idx_v, *scratch):
        bufs = scratch[:_NBUF]
        gsems = scratch[_NBUF:2 * _NBUF]
        osems = scratch[2 * _NBUF:]
        wid = lax.axis_index("s") * _NC + lax.axis_index("c")
        base = wid * b_per_w
        pltpu.sync_copy(idx_hbm.at[pl.ds(base, b_per_w)], idx_v)

        def csize(g):
            return min(_CHUNK, b_per_w - g * _CHUNK)

        def start_gather(g, b):
            return pltpu.async_copy(
                table_hbm.at[idx_v.at[pl.ds(g * _CHUNK, csize(g))]],
                bufs[b].at[pl.ds(0, csize(g))], gsems[b])

        gathers = [None] * _NBUF
        outs = [None] * _NBUF
        for g in range(min(_NBUF - 1, nchunks)):
            gathers[g] = start_gather(g, g)
        for g in range(nchunks):
            s = g % _NBUF
            fg = g + _NBUF - 1
            if fg < nchunks:
                fs = fg % _NBUF
                if outs[fs] is not None:
                    outs[fs].wait()
                    outs[fs] = None
                gathers[fs] = start_gather(fg, fs)
            gathers[s].wait()
            outs[s] = pltpu.async_copy(
                bufs[s].at[pl.ds(0, csize(g))],
                out_hbm.at[pl.ds(base + g * _CHUNK, csize(g))], osems[s])
        for o in outs:
            if o is not None:
                o.wait()

    return gather_kernel(embedding, idx_flat)


def kernel(x, embedding):
    B, S = x.shape
    V, D = embedding.shape
    idx_flat = x.reshape(B * S).astype(jnp.int32)
    out = _sc_gather(embedding, idx_flat, B * S, D)
    return out.reshape(B, S, D)


# speedup vs baseline: 1.0018x; 1.0018x over previous
import functools

import jax
import jax.numpy as jnp
from jax import lax
from jax.experimental import pallas as pl
from jax.experimental.pallas import tpu as pltpu
from jax.experimental.pallas import tpu_sc as plsc

_NC = 2
_NS = 16
_NW = _NC * _NS

_CHUNK = 16
_NBUF = 7


@functools.partial(jax.jit, static_argnums=(2, 3))
def _sc_gather(embedding, idx_flat, N, D):
    b_per_w = N // _NW
    nchunks = b_per_w // _CHUNK
    mesh = plsc.VectorSubcoreMesh(core_axis_name="c", subcore_axis_name="s")

    @functools.partial(
        pl.kernel,
        out_type=jax.ShapeDtypeStruct((N, D), jnp.float32),
        mesh=mesh,
        scratch_types=(
            [pltpu.VMEM((b_per_w,), jnp.int32)]
            + [pltpu.VMEM((_CHUNK, D), jnp.float32) for _ in range(_NBUF)]
            + [pltpu.SemaphoreType.DMA for _ in range(2 * _NBUF)]
        ),
    )
    def gather_kernel(table_hbm, idx_hbm, out_hbm, idx_v, *scratch):
        bufs = scratch[:_NBUF]
        gsems = scratch[_NBUF:2 * _NBUF]
        osems = scratch[2 * _NBUF:]
        wid = lax.axis_index("s") * _NC + lax.axis_index("c")
        base = wid * b_per_w
        pltpu.sync_copy(idx_hbm.at[pl.ds(base, b_per_w)], idx_v)

        def start_gather(g, b):
            return pltpu.async_copy(
                table_hbm.at[idx_v.at[pl.ds(g * _CHUNK, _CHUNK)]],
                bufs[b], gsems[b])

        gathers = [None] * _NBUF
        outs = [None] * _NBUF
        for g in range(min(_NBUF - 1, nchunks)):
            gathers[g] = start_gather(g, g)
        for g in range(nchunks):
            s = g % _NBUF
            fg = g + _NBUF - 1
            if fg < nchunks:
                fs = fg % _NBUF
                if outs[fs] is not None:
                    outs[fs].wait()
                    outs[fs] = None
                gathers[fs] = start_gather(fg, fs)
            gathers[s].wait()
            outs[s] = pltpu.async_copy(
                bufs[s], out_hbm.at[pl.ds(base + g * _CHUNK, _CHUNK)], osems[s])
        for o in outs:
            if o is not None:
                o.wait()

    return gather_kernel(embedding, idx_flat)


def kernel(x, embedding):
    B, S = x.shape
    V, D = embedding.shape
    idx_flat = x.reshape(B * S).astype(jnp.int32)
    out = _sc_gather(embedding, idx_flat, B * S, D)
    return out.reshape(B, S, D)
